# Initial kernel scaffold; baseline (speedup 1.0000x reference)
import functools

import jax
import jax.numpy as jnp
from jax import lax
from jax.experimental import pallas as pl
from jax.experimental.pallas import tpu as pltpu
from jax.experimental.pallas import tpu_sc as plsc

_NW = 32
_CH = 128


def _sc_gather(node_states, src2):
    nchunk, ch = src2.shape
    e_total = nchunk * ch
    d = node_states.shape[1]
    mesh = plsc.VectorSubcoreMesh(core_axis_name="c", subcore_axis_name="s")

    @functools.partial(
        pl.kernel,
        mesh=mesh,
        out_type=jax.ShapeDtypeStruct((e_total, d), jnp.float32),
        scratch_types=[
            pltpu.VMEM((ch,), jnp.int32),
            pltpu.VMEM((ch, d), jnp.float32),
            pltpu.SemaphoreType.DMA,
        ],
    )
    def k(ns_hbm, src_hbm, out_hbm, idx_v, rows_v, sem):
        wid = lax.axis_index("s") * 2 + lax.axis_index("c")

        @pl.loop(wid, nchunk, step=_NW)
        def _(j):
            pltpu.sync_copy(src_hbm.at[j], idx_v)
            pltpu.async_copy(ns_hbm.at[idx_v], rows_v, sem).wait()
            pltpu.sync_copy(rows_v, out_hbm.at[pl.ds(j * ch, ch)])

    return k(node_states, src2)


def _tc_messages(x_i, a2):
    e_total, dd = a2.shape
    d = x_i.shape[1]
    blk = 3200
    grid = e_total // blk

    def body(x_ref, a_ref, o_ref):
        x = x_ref[...]
        a = a_ref[...]
        row = lax.broadcasted_iota(jnp.int32, (d, dd), 0)
        col = lax.broadcasted_iota(jnp.int32, (d, dd), 1)
        s_mat = (col // d == row).astype(jnp.float32)
        xb = lax.dot(x, s_mat, precision=lax.Precision.HIGHEST)
        p = xb * a
        colt = lax.broadcasted_iota(jnp.int32, (dd, d), 0)
        kt = lax.broadcasted_iota(jnp.int32, (dd, d), 1)
        t_mat = (colt % d == kt).astype(jnp.float32)
        o_ref[...] = lax.dot(p, t_mat, precision=lax.Precision.HIGHEST)

    return pl.pallas_call(
        body,
        grid=(grid,),
        in_specs=[
            pl.BlockSpec((blk, d), lambda i: (i, 0)),
            pl.BlockSpec((blk, dd), lambda i: (i, 0)),
        ],
        out_specs=pl.BlockSpec((blk, d), lambda i: (i, 0)),
        out_shape=jax.ShapeDtypeStruct((e_total, d), jnp.float32),
    )(x_i, a2)


def _sc_scatter(msg, dst2, n_nodes):
    nchunk, ch = dst2.shape
    d = msg.shape[1]
    zb = 80
    nzc = n_nodes // zb
    mesh = plsc.VectorSubcoreMesh(core_axis_name="c", subcore_axis_name="s")

    @functools.partial(
        pl.kernel,
        mesh=mesh,
        out_type=[
            jax.ShapeDtypeStruct((2 * n_nodes, d), jnp.float32),
            jax.ShapeDtypeStruct((2 * n_nodes, d), jnp.float32),
        ],
        scratch_types=[
            pltpu.VMEM((ch,), jnp.int32),
            pltpu.VMEM((ch, d), jnp.float32),
            pltpu.VMEM((ch, d), jnp.float32),
            pltpu.VMEM((80, d), jnp.float32),
            pltpu.VMEM_SHARED((n_nodes, d), jnp.float32),
            pltpu.VMEM_SHARED((n_nodes, d), jnp.float32),
        ],
    )
    def k(msg_hbm, dst_hbm, sums_hbm, cnts_hbm,
          idx_v, msg_v, ones_v, zero_v, acc_sh, cnt_sh):
        core = lax.axis_index("c")
        sid = lax.axis_index("s")
        wid = sid * 2 + core

        @pl.loop(0, ch)
        def _(i):
            ones_v[i] = jnp.ones((d,), jnp.float32)

        @pl.loop(0, zb)
        def _(i):
            zero_v[i] = jnp.zeros((d,), jnp.float32)

        @pl.loop(sid, nzc, step=16)
        def _(c):
            pltpu.sync_copy(zero_v, acc_sh.at[pl.ds(c * zb, zb)])
            pltpu.sync_copy(zero_v, cnt_sh.at[pl.ds(c * zb, zb)])

        plsc.subcore_barrier()

        @pl.loop(wid, nchunk, step=_NW)
        def _(j):
            pltpu.sync_copy(dst_hbm.at[j], idx_v)
            pltpu.sync_copy(msg_hbm.at[pl.ds(j * ch, ch)], msg_v)
            pltpu.sync_copy(msg_v, acc_sh.at[idx_v], add=True)
            pltpu.sync_copy(ones_v, cnt_sh.at[idx_v], add=True)

        plsc.subcore_barrier()

        @pl.loop(sid, nzc, step=16)
        def _(c):
            base = core * n_nodes + c * zb
            pltpu.sync_copy(acc_sh.at[pl.ds(c * zb, zb)],
                            sums_hbm.at[pl.ds(base, zb)])
            pltpu.sync_copy(cnt_sh.at[pl.ds(c * zb, zb)],
                            cnts_hbm.at[pl.ds(base, zb)])

    return k(msg, dst2)


def _tc_gru(node_states, sums, cnts, w_ih, w_hh, b_ih, b_hh):
    n, d = node_states.shape
    blk = 2000
    grid = n // blk
    nb = n // blk

    def body(h_ref, s0_ref, s1_ref, c0_ref, c1_ref,
             wih_ref, whh_ref, bih_ref, bhh_ref, o_ref):
        s = s0_ref[...] + s1_ref[...]
        c = c0_ref[...] + c1_ref[...]
        mean = s / jnp.maximum(c, 1.0)
        h = h_ref[...]
        dims = (((1,), (1,)), ((), ()))
        gx = lax.dot_general(mean, wih_ref[...], dims,
                             precision=lax.Precision.HIGHEST) + bih_ref[0]
        gh = lax.dot_general(h, whh_ref[...], dims,
                             precision=lax.Precision.HIGHEST) + bhh_ref[0]
        r = jax.nn.sigmoid(gx[:, :d] + gh[:, :d])
        z = jax.nn.sigmoid(gx[:, d:2 * d] + gh[:, d:2 * d])
        nn = jnp.tanh(gx[:, 2 * d:] + r * gh[:, 2 * d:])
        o_ref[...] = (1.0 - z) * nn + z * h

    return pl.pallas_call(
        body,
        grid=(grid,),
        in_specs=[
            pl.BlockSpec((blk, d), lambda i: (i, 0)),
            pl.BlockSpec((blk, d), lambda i: (i, 0)),
            pl.BlockSpec((blk, d), lambda i, _nb=nb: (i + _nb, 0)),
            pl.BlockSpec((blk, d), lambda i: (i, 0)),
            pl.BlockSpec((blk, d), lambda i, _nb=nb: (i + _nb, 0)),
            pl.BlockSpec((3 * d, d), lambda i: (0, 0)),
            pl.BlockSpec((3 * d, d), lambda i: (0, 0)),
            pl.BlockSpec((1, 3 * d), lambda i: (0, 0)),
            pl.BlockSpec((1, 3 * d), lambda i: (0, 0)),
        ],
        out_specs=pl.BlockSpec((blk, d), lambda i: (i, 0)),
        out_shape=jax.ShapeDtypeStruct((n, d), jnp.float32),
    )(node_states, sums, sums, cnts, cnts,
      w_ih, w_hh, b_ih.reshape(1, 3 * d), b_hh.reshape(1, 3 * d))


def kernel(node_states, edge_index, a_in, w_ih, w_hh, b_ih, b_hh):
    e_total = edge_index.shape[0]
    n, d = node_states.shape
    src2 = edge_index[:, 0].reshape(e_total // _CH, _CH)
    dst2 = edge_index[:, 1].reshape(e_total // _CH, _CH)
    x_i = _sc_gather(node_states, src2)
    a2 = a_in.reshape(e_total, d * d)
    msg = _tc_messages(x_i, a2)
    sums, cnts = _sc_scatter(msg, dst2, n)
    return _tc_gru(node_states, sums, cnts, w_ih, w_hh, b_ih, b_hh)


# speedup vs baseline: 2.3325x; 2.3325x over previous
import functools

import jax
import jax.numpy as jnp
from jax import lax
from jax.experimental import pallas as pl
from jax.experimental.pallas import tpu as pltpu
from jax.experimental.pallas import tpu_sc as plsc

_NW = 32
_CH = 128


def _sc_gather(node_states, src2):
    nchunk, ch = src2.shape
    e_total = nchunk * ch
    d = node_states.shape[1]
    mesh = plsc.VectorSubcoreMesh(core_axis_name="c", subcore_axis_name="s")

    @functools.partial(
        pl.kernel,
        mesh=mesh,
        out_type=jax.ShapeDtypeStruct((e_total, d), jnp.float32),
        compiler_params=pltpu.CompilerParams(use_tc_tiling_on_sc=False),
        scratch_types=[
            pltpu.VMEM((ch,), jnp.int32),
            pltpu.VMEM((ch, d), jnp.float32),
            pltpu.SemaphoreType.DMA,
        ],
    )
    def k(ns_hbm, src_hbm, out_hbm, idx_v, rows_v, sem):
        wid = lax.axis_index("s") * 2 + lax.axis_index("c")

        @pl.loop(wid, nchunk, step=_NW)
        def _(j):
            pltpu.sync_copy(src_hbm.at[j], idx_v)
            pltpu.async_copy(ns_hbm.at[idx_v], rows_v, sem).wait()
            pltpu.sync_copy(rows_v, out_hbm.at[pl.ds(j * ch, ch)])

    return k(node_states, src2)


def _tc_messages(x_i, a2):
    e_total, dd = a2.shape
    d = x_i.shape[1]
    blk = 3200
    grid = e_total // blk

    def body(x_ref, a_ref, o_ref):
        x = x_ref[...]
        a = a_ref[...]
        row = lax.broadcasted_iota(jnp.int32, (d, dd), 0)
        col = lax.broadcasted_iota(jnp.int32, (d, dd), 1)
        s_mat = (col // d == row).astype(jnp.float32)
        xb = lax.dot(x, s_mat, precision=lax.Precision.HIGHEST)
        p = xb * a
        colt = lax.broadcasted_iota(jnp.int32, (dd, d), 0)
        kt = lax.broadcasted_iota(jnp.int32, (dd, d), 1)
        t_mat = (colt % d == kt).astype(jnp.float32)
        o_ref[...] = lax.dot(p, t_mat, precision=lax.Precision.HIGHEST)

    return pl.pallas_call(
        body,
        grid=(grid,),
        in_specs=[
            pl.BlockSpec((blk, d), lambda i: (i, 0)),
            pl.BlockSpec((blk, dd), lambda i: (i, 0)),
        ],
        out_specs=pl.BlockSpec((blk, d), lambda i: (i, 0)),
        out_shape=jax.ShapeDtypeStruct((e_total, d), jnp.float32),
    )(x_i, a2)


def _sc_scatter(msg, dst2, n_nodes):
    nchunk, ch = dst2.shape
    d = msg.shape[1]
    zb = 80
    nzc = n_nodes // zb
    mesh = plsc.VectorSubcoreMesh(core_axis_name="c", subcore_axis_name="s")

    @functools.partial(
        pl.kernel,
        mesh=mesh,
        out_type=[
            jax.ShapeDtypeStruct((2 * n_nodes, d), jnp.float32),
            jax.ShapeDtypeStruct((2 * n_nodes, d), jnp.float32),
        ],
        compiler_params=pltpu.CompilerParams(use_tc_tiling_on_sc=False),
        scratch_types=[
            pltpu.VMEM((ch,), jnp.int32),
            pltpu.VMEM((ch, d), jnp.float32),
            pltpu.VMEM((ch, d), jnp.float32),
            pltpu.VMEM((80, d), jnp.float32),
            pltpu.VMEM_SHARED((n_nodes, d), jnp.float32),
            pltpu.VMEM_SHARED((n_nodes, d), jnp.float32),
        ],
    )
    def k(msg_hbm, dst_hbm, sums_hbm, cnts_hbm,
          idx_v, msg_v, ones_v, zero_v, acc_sh, cnt_sh):
        core = lax.axis_index("c")
        sid = lax.axis_index("s")
        wid = sid * 2 + core

        @pl.loop(0, ch)
        def _(i):
            ones_v[i] = jnp.ones((d,), jnp.float32)

        @pl.loop(0, zb)
        def _(i):
            zero_v[i] = jnp.zeros((d,), jnp.float32)

        @pl.loop(sid, nzc, step=16)
        def _(c):
            pltpu.sync_copy(zero_v, acc_sh.at[pl.ds(c * zb, zb)])
            pltpu.sync_copy(zero_v, cnt_sh.at[pl.ds(c * zb, zb)])

        plsc.subcore_barrier()

        @pl.loop(wid, nchunk, step=_NW)
        def _(j):
            pltpu.sync_copy(dst_hbm.at[j], idx_v)
            pltpu.sync_copy(msg_hbm.at[pl.ds(j * ch, ch)], msg_v)
            pltpu.sync_copy(msg_v, acc_sh.at[idx_v], add=True)
            pltpu.sync_copy(ones_v, cnt_sh.at[idx_v], add=True)

        plsc.subcore_barrier()

        @pl.loop(sid, nzc, step=16)
        def _(c):
            base = core * n_nodes + c * zb
            pltpu.sync_copy(acc_sh.at[pl.ds(c * zb, zb)],
                            sums_hbm.at[pl.ds(base, zb)])
            pltpu.sync_copy(cnt_sh.at[pl.ds(c * zb, zb)],
                            cnts_hbm.at[pl.ds(base, zb)])

    return k(msg, dst2)


def _tc_gru(node_states, sums, cnts, w_ih, w_hh, b_ih, b_hh):
    n, d = node_states.shape
    blk = 2000
    grid = n // blk
    nb = n // blk

    def body(h_ref, s0_ref, s1_ref, c0_ref, c1_ref,
             wih_ref, whh_ref, bih_ref, bhh_ref, o_ref):
        s = s0_ref[...] + s1_ref[...]
        c = c0_ref[...] + c1_ref[...]
        mean = s / jnp.maximum(c, 1.0)
        h = h_ref[...]
        dims = (((1,), (1,)), ((), ()))
        gx = lax.dot_general(mean, wih_ref[...], dims,
                             precision=lax.Precision.HIGHEST) + bih_ref[0]
        gh = lax.dot_general(h, whh_ref[...], dims,
                             precision=lax.Precision.HIGHEST) + bhh_ref[0]
        r = jax.nn.sigmoid(gx[:, :d] + gh[:, :d])
        z = jax.nn.sigmoid(gx[:, d:2 * d] + gh[:, d:2 * d])
        nn = jnp.tanh(gx[:, 2 * d:] + r * gh[:, 2 * d:])
        o_ref[...] = (1.0 - z) * nn + z * h

    return pl.pallas_call(
        body,
        grid=(grid,),
        in_specs=[
            pl.BlockSpec((blk, d), lambda i: (i, 0)),
            pl.BlockSpec((blk, d), lambda i: (i, 0)),
            pl.BlockSpec((blk, d), lambda i, _nb=nb: (i + _nb, 0)),
            pl.BlockSpec((blk, d), lambda i: (i, 0)),
            pl.BlockSpec((blk, d), lambda i, _nb=nb: (i + _nb, 0)),
            pl.BlockSpec((3 * d, d), lambda i: (0, 0)),
            pl.BlockSpec((3 * d, d), lambda i: (0, 0)),
            pl.BlockSpec((1, 3 * d), lambda i: (0, 0)),
            pl.BlockSpec((1, 3 * d), lambda i: (0, 0)),
        ],
        out_specs=pl.BlockSpec((blk, d), lambda i: (i, 0)),
        out_shape=jax.ShapeDtypeStruct((n, d), jnp.float32),
    )(node_states, sums, sums, cnts, cnts,
      w_ih, w_hh, b_ih.reshape(1, 3 * d), b_hh.reshape(1, 3 * d))


def kernel(node_states, edge_index, a_in, w_ih, w_hh, b_ih, b_hh):
    e_total = edge_index.shape[0]
    n, d = node_states.shape
    src2 = edge_index[:, 0].reshape(e_total // _CH, _CH)
    dst2 = edge_index[:, 1].reshape(e_total // _CH, _CH)
    x_i = _sc_gather(node_states, src2)
    a2 = a_in.reshape(e_total, d * d)
    msg = _tc_messages(x_i, a2)
    sums, cnts = _sc_scatter(msg, dst2, n)
    return _tc_gru(node_states, sums, cnts, w_ih, w_hh, b_ih, b_hh)


# speedup vs baseline: 4.0252x; 1.7257x over previous
import functools

import jax
import jax.numpy as jnp
from jax import lax
from jax.experimental import pallas as pl
from jax.experimental.pallas import tpu as pltpu
from jax.experimental.pallas import tpu_sc as plsc

_NW = 32
_CH = 128


def _sc_gather(node_states, src2):
    nchunk, ch = src2.shape
    e_total = nchunk * ch
    d = node_states.shape[1]
    mesh = plsc.VectorSubcoreMesh(core_axis_name="c", subcore_axis_name="s")

    @functools.partial(
        pl.kernel,
        mesh=mesh,
        out_type=jax.ShapeDtypeStruct((e_total, d), jnp.float32),
        compiler_params=pltpu.CompilerParams(use_tc_tiling_on_sc=False),
        scratch_types=[
            pltpu.VMEM((ch,), jnp.int32),
            pltpu.VMEM((ch, d), jnp.float32),
            pltpu.SemaphoreType.DMA,
        ],
    )
    def k(ns_hbm, src_hbm, out_hbm, idx_v, rows_v, sem):
        wid = lax.axis_index("s") * 2 + lax.axis_index("c")

        @pl.loop(wid, nchunk, step=_NW)
        def _(j):
            pltpu.sync_copy(src_hbm.at[j], idx_v)
            pltpu.async_copy(ns_hbm.at[idx_v], rows_v, sem).wait()
            pltpu.sync_copy(rows_v, out_hbm.at[pl.ds(j * ch, ch)])

    return k(node_states, src2)


def _tc_messages(x_i, a2):
    e_total, dd = a2.shape
    d = x_i.shape[1]
    blk = 3200
    grid = e_total // blk

    def body(x_ref, a_ref, o_ref):
        x = x_ref[...]
        a = a_ref[...]
        row = lax.broadcasted_iota(jnp.int32, (d, dd), 0)
        col = lax.broadcasted_iota(jnp.int32, (d, dd), 1)
        s_mat = (col // d == row).astype(jnp.bfloat16)
        xh = x.astype(jnp.bfloat16)
        xl = (x - xh.astype(jnp.float32)).astype(jnp.bfloat16)
        xb = (lax.dot(xh, s_mat, preferred_element_type=jnp.float32) +
              lax.dot(xl, s_mat, preferred_element_type=jnp.float32))
        p = xb * a
        q = p[:, :128] + p[:, 128:]
        q = q[:, :64] + q[:, 64:]
        q = q[:, :32] + q[:, 32:]
        o_ref[...] = q[:, :d] + q[:, d:]

    return pl.pallas_call(
        body,
        grid=(grid,),
        in_specs=[
            pl.BlockSpec((blk, d), lambda i: (i, 0)),
            pl.BlockSpec((blk, dd), lambda i: (i, 0)),
        ],
        out_specs=pl.BlockSpec((blk, d), lambda i: (i, 0)),
        out_shape=jax.ShapeDtypeStruct((e_total, d), jnp.float32),
    )(x_i, a2)


def _sc_scatter(msg, dst2, n_nodes):
    nchunk, ch = dst2.shape
    d = msg.shape[1]
    zb = 80
    nzc = n_nodes // zb
    mesh = plsc.VectorSubcoreMesh(core_axis_name="c", subcore_axis_name="s")

    @functools.partial(
        pl.kernel,
        mesh=mesh,
        out_type=[
            jax.ShapeDtypeStruct((2 * n_nodes, d), jnp.float32),
            jax.ShapeDtypeStruct((2 * n_nodes, d), jnp.float32),
        ],
        compiler_params=pltpu.CompilerParams(use_tc_tiling_on_sc=False),
        scratch_types=[
            pltpu.VMEM((ch,), jnp.int32),
            pltpu.VMEM((ch, d), jnp.float32),
            pltpu.VMEM((ch, d), jnp.float32),
            pltpu.VMEM((80, d), jnp.float32),
            pltpu.VMEM_SHARED((n_nodes, d), jnp.float32),
            pltpu.VMEM_SHARED((n_nodes, d), jnp.float32),
        ],
    )
    def k(msg_hbm, dst_hbm, sums_hbm, cnts_hbm,
          idx_v, msg_v, ones_v, zero_v, acc_sh, cnt_sh):
        core = lax.axis_index("c")
        sid = lax.axis_index("s")
        wid = sid * 2 + core

        @pl.loop(0, ch)
        def _(i):
            ones_v[i] = jnp.ones((d,), jnp.float32)

        @pl.loop(0, zb)
        def _(i):
            zero_v[i] = jnp.zeros((d,), jnp.float32)

        @pl.loop(sid, nzc, step=16)
        def _(c):
            pltpu.sync_copy(zero_v, acc_sh.at[pl.ds(c * zb, zb)])
            pltpu.sync_copy(zero_v, cnt_sh.at[pl.ds(c * zb, zb)])

        plsc.subcore_barrier()

        @pl.loop(wid, nchunk, step=_NW)
        def _(j):
            pltpu.sync_copy(dst_hbm.at[j], idx_v)
            pltpu.sync_copy(msg_hbm.at[pl.ds(j * ch, ch)], msg_v)
            pltpu.sync_copy(msg_v, acc_sh.at[idx_v], add=True)
            pltpu.sync_copy(ones_v, cnt_sh.at[idx_v], add=True)

        plsc.subcore_barrier()

        @pl.loop(sid, nzc, step=16)
        def _(c):
            base = core * n_nodes + c * zb
            pltpu.sync_copy(acc_sh.at[pl.ds(c * zb, zb)],
                            sums_hbm.at[pl.ds(base, zb)])
            pltpu.sync_copy(cnt_sh.at[pl.ds(c * zb, zb)],
                            cnts_hbm.at[pl.ds(base, zb)])

    return k(msg, dst2)


def _tc_gru(node_states, sums, cnts, w_ih, w_hh, b_ih, b_hh):
    n, d = node_states.shape
    blk = 2000
    grid = n // blk
    nb = n // blk

    def body(h_ref, s0_ref, s1_ref, c0_ref, c1_ref,
             wih_ref, whh_ref, bih_ref, bhh_ref, o_ref):
        s = s0_ref[...] + s1_ref[...]
        c = c0_ref[...] + c1_ref[...]
        mean = s / jnp.maximum(c, 1.0)
        h = h_ref[...]
        dims = (((1,), (1,)), ((), ()))
        gx = lax.dot_general(mean, wih_ref[...], dims,
                             precision=lax.Precision.HIGHEST) + bih_ref[0]
        gh = lax.dot_general(h, whh_ref[...], dims,
                             precision=lax.Precision.HIGHEST) + bhh_ref[0]
        r = jax.nn.sigmoid(gx[:, :d] + gh[:, :d])
        z = jax.nn.sigmoid(gx[:, d:2 * d] + gh[:, d:2 * d])
        nn = jnp.tanh(gx[:, 2 * d:] + r * gh[:, 2 * d:])
        o_ref[...] = (1.0 - z) * nn + z * h

    return pl.pallas_call(
        body,
        grid=(grid,),
        in_specs=[
            pl.BlockSpec((blk, d), lambda i: (i, 0)),
            pl.BlockSpec((blk, d), lambda i: (i, 0)),
            pl.BlockSpec((blk, d), lambda i, _nb=nb: (i + _nb, 0)),
            pl.BlockSpec((blk, d), lambda i: (i, 0)),
            pl.BlockSpec((blk, d), lambda i, _nb=nb: (i + _nb, 0)),
            pl.BlockSpec((3 * d, d), lambda i: (0, 0)),
            pl.BlockSpec((3 * d, d), lambda i: (0, 0)),
            pl.BlockSpec((1, 3 * d), lambda i: (0, 0)),
            pl.BlockSpec((1, 3 * d), lambda i: (0, 0)),
        ],
        out_specs=pl.BlockSpec((blk, d), lambda i: (i, 0)),
        out_shape=jax.ShapeDtypeStruct((n, d), jnp.float32),
    )(node_states, sums, sums, cnts, cnts,
      w_ih, w_hh, b_ih.reshape(1, 3 * d), b_hh.reshape(1, 3 * d))


def kernel(node_states, edge_index, a_in, w_ih, w_hh, b_ih, b_hh):
    e_total = edge_index.shape[0]
    n, d = node_states.shape
    src2 = edge_index[:, 0].reshape(e_total // _CH, _CH)
    dst2 = edge_index[:, 1].reshape(e_total // _CH, _CH)
    x_i = _sc_gather(node_states, src2)
    a2 = a_in.reshape(e_total, d * d)
    msg = _tc_messages(x_i, a2)
    sums, cnts = _sc_scatter(msg, dst2, n)
    return _tc_gru(node_states, sums, cnts, w_ih, w_hh, b_ih, b_hh)


# speedup vs baseline: 4.5180x; 1.1224x over previous
import functools

import jax
import jax.numpy as jnp
from jax import lax
from jax.experimental import pallas as pl
from jax.experimental.pallas import tpu as pltpu
from jax.experimental.pallas import tpu_sc as plsc

_NW = 32
_CH = 128


def _sc_gather(node_states, src2, dst2, n_nodes):
    nchunk, ch = src2.shape
    e_total = nchunk * ch
    d = node_states.shape[1]
    base_c = nchunk // _NW
    rem = nchunk % _NW
    maxc = base_c + (1 if rem else 0)
    zb = 80
    nzc = n_nodes // zb
    mesh = plsc.VectorSubcoreMesh(core_axis_name="c", subcore_axis_name="s")

    @functools.partial(
        pl.kernel,
        mesh=mesh,
        out_type=[
            jax.ShapeDtypeStruct((e_total, d), jnp.float32),
            jax.ShapeDtypeStruct((2 * n_nodes, d), jnp.float32),
        ],
        compiler_params=pltpu.CompilerParams(use_tc_tiling_on_sc=False),
        scratch_types=[
            pltpu.VMEM((maxc, ch), jnp.int32),
            pltpu.VMEM((maxc, ch), jnp.int32),
            pltpu.VMEM((maxc * ch, d), jnp.float32),
            pltpu.VMEM((ch, d), jnp.float32),
            pltpu.VMEM((zb, d), jnp.float32),
            pltpu.VMEM_SHARED((n_nodes, d), jnp.float32),
            pltpu.SemaphoreType.DMA,
        ],
    )
    def k(ns_hbm, src_hbm, dst_hbm, out_hbm, cnts_hbm,
          sidx, didx, rows_all, ones_v, zero_v, cnt_sh, sem):
        core = lax.axis_index("c")
        sid = lax.axis_index("s")
        wid = sid * 2 + core
        startc = base_c * wid + jnp.minimum(wid, rem)

        @pl.loop(0, ch)
        def _(i):
            ones_v[i] = jnp.ones((d,), jnp.float32)

        @pl.loop(0, zb)
        def _(i):
            zero_v[i] = jnp.zeros((d,), jnp.float32)

        @pl.loop(sid, nzc, step=16)
        def _(c):
            pltpu.sync_copy(zero_v, cnt_sh.at[pl.ds(c * zb, zb)])

        plsc.subcore_barrier()

        def flow(cnt):
            pltpu.sync_copy(src_hbm.at[pl.ds(startc, cnt)],
                            sidx.at[pl.ds(0, cnt)])
            pltpu.sync_copy(dst_hbm.at[pl.ds(startc, cnt)],
                            didx.at[pl.ds(0, cnt)])

            @pl.loop(0, cnt)
            def _(j):
                pltpu.async_copy(ns_hbm.at[sidx.at[j]],
                                 rows_all.at[pl.ds(j * ch, ch)], sem)

            @pl.loop(0, cnt)
            def _(j):
                pltpu.sync_copy(ones_v, cnt_sh.at[didx.at[j]], add=True)

            pltpu.make_async_copy(ns_hbm.at[pl.ds(0, cnt * ch)],
                                  rows_all.at[pl.ds(0, cnt * ch)], sem).wait()
            pltpu.sync_copy(rows_all.at[pl.ds(0, cnt * ch)],
                            out_hbm.at[pl.ds(startc * ch, cnt * ch)])

        if rem:
            @pl.when(wid < rem)
            def _():
                flow(base_c + 1)

            @pl.when(wid >= rem)
            def _():
                flow(base_c)
        else:
            flow(base_c)

        plsc.subcore_barrier()

        @pl.loop(sid, nzc, step=16)
        def _(c):
            pltpu.sync_copy(cnt_sh.at[pl.ds(c * zb, zb)],
                            cnts_hbm.at[pl.ds(core * n_nodes + c * zb, zb)])

    return k(node_states, src2, dst2)


def _tc_messages(x_i, a2):
    e_total, dd = a2.shape
    d = x_i.shape[1]
    blk = 3200
    grid = e_total // blk

    def body(x_ref, a_ref, o_ref):
        x = x_ref[...]
        a = a_ref[...]
        row = lax.broadcasted_iota(jnp.int32, (d, dd), 0)
        col = lax.broadcasted_iota(jnp.int32, (d, dd), 1)
        s_mat = (col // d == row).astype(jnp.bfloat16)
        xh = x.astype(jnp.bfloat16)
        xl = (x - xh.astype(jnp.float32)).astype(jnp.bfloat16)
        xb = (lax.dot(xh, s_mat, preferred_element_type=jnp.float32) +
              lax.dot(xl, s_mat, preferred_element_type=jnp.float32))
        p = xb * a
        q = p[:, :128] + p[:, 128:]
        q = q[:, :64] + q[:, 64:]
        q = q[:, :32] + q[:, 32:]
        o_ref[...] = q[:, :d] + q[:, d:]

    return pl.pallas_call(
        body,
        grid=(grid,),
        in_specs=[
            pl.BlockSpec((blk, d), lambda i: (i, 0)),
            pl.BlockSpec((blk, dd), lambda i: (i, 0)),
        ],
        out_specs=pl.BlockSpec((blk, d), lambda i: (i, 0)),
        out_shape=jax.ShapeDtypeStruct((e_total, d), jnp.float32),
    )(x_i, a2)


def _sc_scatter(msg, dst2, n_nodes):
    nchunk, ch = dst2.shape
    d = msg.shape[1]
    base_c = nchunk // _NW
    rem = nchunk % _NW
    maxc = base_c + (1 if rem else 0)
    zb = 80
    nzc = n_nodes // zb
    mesh = plsc.VectorSubcoreMesh(core_axis_name="c", subcore_axis_name="s")

    @functools.partial(
        pl.kernel,
        mesh=mesh,
        out_type=jax.ShapeDtypeStruct((2 * n_nodes, d), jnp.float32),
        compiler_params=pltpu.CompilerParams(use_tc_tiling_on_sc=False),
        scratch_types=[
            pltpu.VMEM((maxc, ch), jnp.int32),
            pltpu.VMEM((maxc * ch, d), jnp.float32),
            pltpu.VMEM((zb, d), jnp.float32),
            pltpu.VMEM_SHARED((n_nodes, d), jnp.float32),
            pltpu.SemaphoreType.DMA,
        ],
    )
    def k(msg_hbm, dst_hbm, sums_hbm, didx, msg_all, zero_v, acc_sh, sem):
        core = lax.axis_index("c")
        sid = lax.axis_index("s")
        wid = sid * 2 + core
        startc = base_c * wid + jnp.minimum(wid, rem)

        @pl.loop(0, zb)
        def _(i):
            zero_v[i] = jnp.zeros((d,), jnp.float32)

        @pl.loop(sid, nzc, step=16)
        def _(c):
            pltpu.sync_copy(zero_v, acc_sh.at[pl.ds(c * zb, zb)])

        plsc.subcore_barrier()

        def flow(cnt):
            load = pltpu.async_copy(
                msg_hbm.at[pl.ds(startc * ch, cnt * ch)],
                msg_all.at[pl.ds(0, cnt * ch)], sem)
            pltpu.sync_copy(dst_hbm.at[pl.ds(startc, cnt)],
                            didx.at[pl.ds(0, cnt)])
            load.wait()

            @pl.loop(0, cnt)
            def _(j):
                pltpu.sync_copy(msg_all.at[pl.ds(j * ch, ch)],
                                acc_sh.at[didx.at[j]], add=True)

        if rem:
            @pl.when(wid < rem)
            def _():
                flow(base_c + 1)

            @pl.when(wid >= rem)
            def _():
                flow(base_c)
        else:
            flow(base_c)

        plsc.subcore_barrier()

        @pl.loop(sid, nzc, step=16)
        def _(c):
            pltpu.sync_copy(acc_sh.at[pl.ds(c * zb, zb)],
                            sums_hbm.at[pl.ds(core * n_nodes + c * zb, zb)])

    return k(msg, dst2)


def _tc_gru(node_states, sums, cnts, w_ih, w_hh, b_ih, b_hh):
    n, d = node_states.shape
    blk = 2000
    grid = n // blk
    nb = n // blk

    def body(h_ref, s0_ref, s1_ref, c0_ref, c1_ref,
             wih_ref, whh_ref, bih_ref, bhh_ref, o_ref):
        s = s0_ref[...] + s1_ref[...]
        c = c0_ref[...] + c1_ref[...]
        mean = s / jnp.maximum(c, 1.0)
        h = h_ref[...]
        dims = (((1,), (1,)), ((), ()))
        gx = lax.dot_general(mean, wih_ref[...], dims,
                             precision=lax.Precision.HIGHEST) + bih_ref[0]
        gh = lax.dot_general(h, whh_ref[...], dims,
                             precision=lax.Precision.HIGHEST) + bhh_ref[0]
        r = jax.nn.sigmoid(gx[:, :d] + gh[:, :d])
        z = jax.nn.sigmoid(gx[:, d:2 * d] + gh[:, d:2 * d])
        nn = jnp.tanh(gx[:, 2 * d:] + r * gh[:, 2 * d:])
        o_ref[...] = (1.0 - z) * nn + z * h

    return pl.pallas_call(
        body,
        grid=(grid,),
        in_specs=[
            pl.BlockSpec((blk, d), lambda i: (i, 0)),
            pl.BlockSpec((blk, d), lambda i: (i, 0)),
            pl.BlockSpec((blk, d), lambda i, _nb=nb: (i + _nb, 0)),
            pl.BlockSpec((blk, d), lambda i: (i, 0)),
            pl.BlockSpec((blk, d), lambda i, _nb=nb: (i + _nb, 0)),
            pl.BlockSpec((3 * d, d), lambda i: (0, 0)),
            pl.BlockSpec((3 * d, d), lambda i: (0, 0)),
            pl.BlockSpec((1, 3 * d), lambda i: (0, 0)),
            pl.BlockSpec((1, 3 * d), lambda i: (0, 0)),
        ],
        out_specs=pl.BlockSpec((blk, d), lambda i: (i, 0)),
        out_shape=jax.ShapeDtypeStruct((n, d), jnp.float32),
    )(node_states, sums, sums, cnts, cnts,
      w_ih, w_hh, b_ih.reshape(1, 3 * d), b_hh.reshape(1, 3 * d))


def kernel(node_states, edge_index, a_in, w_ih, w_hh, b_ih, b_hh):
    e_total = edge_index.shape[0]
    n, d = node_states.shape
    src2 = edge_index[:, 0].reshape(e_total // _CH, _CH)
    dst2 = edge_index[:, 1].reshape(e_total // _CH, _CH)
    x_i, cnts = _sc_gather(node_states, src2, dst2, n)
    a2 = a_in.reshape(e_total, d * d)
    msg = _tc_messages(x_i, a2)
    sums = _sc_scatter(msg, dst2, n)
    return _tc_gru(node_states, sums, cnts, w_ih, w_hh, b_ih, b_hh)
